# baseline (speedup 1.0000x reference)
import jax
import jax.numpy as jnp
from jax import lax
from jax.experimental import pallas as pl
from jax.experimental.pallas import tpu as pltpu
from jax.experimental.pallas import tpu_sc as plsc

_K = 64
_BB = 8
_L = 16
_U = 8
_FULL_ITERS = 6
_TAIL1_ITERS = 4
_TAIL2_ITERS = 31 - _FULL_ITERS - _TAIL1_ITERS


def _matvec_body(x1_ref, x2_ref, w1_ref, w2_ref, o_ref):
    bb, d, n = x1_ref.shape
    x1 = x1_ref[...].reshape(bb * d, n)
    x2 = x2_ref[...].reshape(bb * d, n)
    s = jax.lax.dot_general(w1_ref[...], x1, (((1,), (0,)), ((), ())),
                            preferred_element_type=jnp.float32)
    s = s + jax.lax.dot_general(w2_ref[...], x2, (((1,), (0,)), ((), ())),
                                preferred_element_type=jnp.float32)
    o_ref[...] = jnp.maximum(s, 0.0)


def _sc_stage2_body(s_hbm, o_hbm, sv, nv, ids):
    n = s_hbm.shape[1]
    nchunk = n // _L
    sent = jnp.int32(nchunk)
    wid = lax.axis_index("s") * 2 + lax.axis_index("c")
    izeros = jnp.zeros((_L,), jnp.int32)

    def one_batch(b):
        pltpu.sync_copy(s_hbm.at[b], sv.at[pl.ds(0, n)])
        sv[pl.ds(n, _L)] = plsc.bitcast(
            jnp.full((_L,), 0x7FFFFFFF, jnp.int32), jnp.float32)

        def bits_of(c):
            return plsc.bitcast(sv[pl.ds(c * _L, _L)], jnp.int32)

        def count_full(mid):
            def grp(g, acc):
                for j in range(_U):
                    acc = acc + (bits_of(g * _U + j) >= mid).astype(jnp.int32)
                return acc

            acc = lax.fori_loop(0, nchunk // _U, grp, izeros)
            return jnp.sum(acc)

        def step_full(_, lohi):
            lo, hi = lohi
            mid = lo + ((hi - lo + 1) >> 1)
            ge = count_full(mid) >= _K
            return jnp.where(ge, mid, lo), jnp.where(ge, hi, mid - 1)

        lo, hi = lax.fori_loop(0, _FULL_ITERS, step_full,
                               (jnp.int32(0), jnp.int32(0x7F7FFFFF)))

        def cgrp(g, carry):
            cnt, abv = carry
            for j in range(_U):
                c = g * _U + j
                v = bits_of(c)
                pc = plsc.all_reduce_population_count(
                    jnp.logical_and(v >= lo, v <= hi))
                ids[cnt] = jnp.int32(c)
                cnt = cnt + jnp.where(pc[0] > 0, 1, 0).astype(jnp.int32)
                abv = abv + (v > hi).astype(jnp.int32)
            return cnt, abv

        cnt1, above_v = lax.fori_loop(0, nchunk // _U, cgrp,
                                      (jnp.int32(0), izeros))
        above1 = jnp.sum(above_v)
        for j in range(_U):
            ids[cnt1 + j] = sent
        hi_c1 = hi
        trips1 = (cnt1 + _U - 1) // _U

        def count_ids(mid, trips, hic, above):
            def grp(g, acc):
                for j in range(_U):
                    v = bits_of(ids[g * _U + j])
                    acc = acc + jnp.logical_and(
                        v >= mid, v <= hic).astype(jnp.int32)
                return acc

            acc = lax.fori_loop(0, trips, grp, izeros)
            return above + jnp.sum(acc)

        def step_t1(_, lohi):
            lo, hi = lohi
            mid = lo + ((hi - lo + 1) >> 1)
            ge = count_ids(mid, trips1, hi_c1, above1) >= _K
            return jnp.where(ge, mid, lo), jnp.where(ge, hi, mid - 1)

        lo, hi = lax.fori_loop(0, _TAIL1_ITERS, step_t1, (lo, hi))

        def rgrp(g, carry):
            cnt, abv = carry
            for j in range(_U):
                cid = ids[g * _U + j]
                v = bits_of(cid)
                pc = plsc.all_reduce_population_count(
                    jnp.logical_and(v >= lo, v <= hi))
                ids[cnt] = cid
                cnt = cnt + jnp.where(pc[0] > 0, 1, 0).astype(jnp.int32)
                abv = abv + jnp.logical_and(
                    v > hi, v <= hi_c1).astype(jnp.int32)
            return cnt, abv

        cnt2, above2_v = lax.fori_loop(0, trips1, rgrp,
                                       (jnp.int32(0), izeros))
        above2 = above1 + jnp.sum(above2_v)
        for j in range(_U):
            ids[cnt2 + j] = sent
        hi_c2 = hi
        trips2 = (cnt2 + _U - 1) // _U

        def step_t2(_, lohi):
            lo, hi = lohi
            mid = lo + ((hi - lo + 1) >> 1)
            ge = count_ids(mid, trips2, hi_c2, above2) >= _K
            return jnp.where(ge, mid, lo), jnp.where(ge, hi, mid - 1)

        lo, hi = lax.fori_loop(0, _TAIL2_ITERS, step_t2, (lo, hi))

        def egrp(g, dacc):
            for j in range(_U):
                c = g * _U + j
                v = sv[pl.ds(c * _L, _L)]
                bb = plsc.bitcast(v, jnp.int32)
                num = jnp.where(bb >= lo, jnp.exp(v), jnp.float32(0.0))
                nv[pl.ds(c * _L, _L)] = num
                dacc = dacc + num
            return dacc

        dacc = lax.fori_loop(0, nchunk // _U, egrp,
                             jnp.zeros((_L,), jnp.float32))
        den_v = jnp.broadcast_to(jnp.sum(dacc), (_L,))
        inv_v = jnp.ones((_L,), jnp.float32) / den_v

        def sgrp(g, carry):
            for j in range(_U):
                c = g * _U + j
                sv[pl.ds(c * _L, _L)] = nv[pl.ds(c * _L, _L)] * inv_v
            return carry

        lax.fori_loop(0, nchunk // _U, sgrp, 0)
        pltpu.sync_copy(sv.at[pl.ds(0, n)], o_hbm.at[b])

    nb = s_hbm.shape[0]
    for r in range(nb // 32):
        one_batch(wid + r * 32)


def _sc_stage2(scores):
    B, N = scores.shape
    mesh = plsc.VectorSubcoreMesh(
        core_axis_name="c", subcore_axis_name="s", num_cores=2,
        num_subcores=16)
    fn = pl.kernel(
        _sc_stage2_body,
        out_type=jax.ShapeDtypeStruct((B, N), jnp.float32),
        mesh=mesh,
        scratch_types=[
            pltpu.VMEM((N + _L,), jnp.float32),
            pltpu.VMEM((N,), jnp.float32),
            pltpu.SMEM((N // _L + _U,), jnp.int32),
        ],
        compiler_params=pltpu.CompilerParams(needs_layout_passes=False),
    )
    return fn(scores)


def kernel(x0, wa):
    B, N, D = x0.shape
    xt = jnp.transpose(x0, (0, 2, 1))
    h = D // 2
    eye = jnp.eye(_BB, dtype=x0.dtype)
    w1 = jnp.kron(eye, wa[:h].reshape(1, h))
    w2 = jnp.kron(eye, wa[h:].reshape(1, h))

    half = B // 2
    outs = []
    for s in range(2):
        off = s * (half // _BB)
        scores = pl.pallas_call(
            _matvec_body,
            grid=(half // _BB,),
            in_specs=[
                pl.BlockSpec((_BB, h, N), lambda i, off=off: (i + off, 0, 0)),
                pl.BlockSpec((_BB, h, N), lambda i, off=off: (i + off, 1, 0)),
                pl.BlockSpec((_BB, _BB * h), lambda i: (0, 0)),
                pl.BlockSpec((_BB, _BB * h), lambda i: (0, 0)),
            ],
            out_specs=pl.BlockSpec((_BB, N), lambda i: (i, 0)),
            out_shape=jax.ShapeDtypeStruct((half, N), jnp.float32),
        )(xt, xt, w1, w2)
        outs.append(_sc_stage2(scores))

    out = jnp.concatenate(outs, axis=0)
    return out.reshape(B, N, 1)


# speedup vs baseline: 1.0160x; 1.0160x over previous
import jax
import jax.numpy as jnp
from jax import lax
from jax.experimental import pallas as pl
from jax.experimental.pallas import tpu as pltpu
from jax.experimental.pallas import tpu_sc as plsc

_K = 64
_BB = 8
_L = 16
_U = 8
_FULL_ITERS = 6
_TAIL1_ITERS = 4
_TAIL2_ITERS = 31 - _FULL_ITERS - _TAIL1_ITERS


def _matvec_body(x_ref, w_ref, o_ref):
    bb, d, n = x_ref.shape
    x = x_ref[...].reshape(bb * d, n)
    w = w_ref[...]
    s = jax.lax.dot_general(w, x, (((1,), (0,)), ((), ())),
                            preferred_element_type=jnp.float32)
    o_ref[...] = jnp.maximum(s, 0.0)


def _sc_stage2_body(s_hbm, o_hbm, sv, nv, ids):
    n = s_hbm.shape[1]
    nchunk = n // _L
    sent = jnp.int32(nchunk)
    wid = lax.axis_index("s") * 2 + lax.axis_index("c")
    izeros = jnp.zeros((_L,), jnp.int32)

    def one_batch(b):
        pltpu.sync_copy(s_hbm.at[b], sv.at[pl.ds(0, n)])
        sv[pl.ds(n, _L)] = plsc.bitcast(
            jnp.full((_L,), 0x7FFFFFFF, jnp.int32), jnp.float32)

        def bits_of(c):
            return plsc.bitcast(sv[pl.ds(c * _L, _L)], jnp.int32)

        def count_full(mid):
            def grp(g, acc):
                for j in range(_U):
                    acc = acc + (bits_of(g * _U + j) >= mid).astype(jnp.int32)
                return acc

            acc = lax.fori_loop(0, nchunk // _U, grp, izeros)
            return jnp.sum(acc)

        def step_full(_, lohi):
            lo, hi = lohi
            mid = lo + ((hi - lo + 1) >> 1)
            ge = count_full(mid) >= _K
            return jnp.where(ge, mid, lo), jnp.where(ge, hi, mid - 1)

        lo, hi = lax.fori_loop(0, _FULL_ITERS, step_full,
                               (jnp.int32(0), jnp.int32(0x7F7FFFFF)))

        def cgrp(g, carry):
            cnt, abv = carry
            for j in range(_U):
                c = g * _U + j
                v = bits_of(c)
                pc = plsc.all_reduce_population_count(
                    jnp.logical_and(v >= lo, v <= hi))
                ids[cnt] = jnp.int32(c)
                cnt = cnt + jnp.where(pc[0] > 0, 1, 0).astype(jnp.int32)
                abv = abv + (v > hi).astype(jnp.int32)
            return cnt, abv

        cnt1, above_v = lax.fori_loop(0, nchunk // _U, cgrp,
                                      (jnp.int32(0), izeros))
        above1 = jnp.sum(above_v)
        for j in range(_U):
            ids[cnt1 + j] = sent
        hi_c1 = hi
        trips1 = (cnt1 + _U - 1) // _U

        def count_ids(mid, trips, hic, above):
            def grp(g, acc):
                for j in range(_U):
                    v = bits_of(ids[g * _U + j])
                    acc = acc + jnp.logical_and(
                        v >= mid, v <= hic).astype(jnp.int32)
                return acc

            acc = lax.fori_loop(0, trips, grp, izeros)
            return above + jnp.sum(acc)

        def step_t1(_, lohi):
            lo, hi = lohi
            mid = lo + ((hi - lo + 1) >> 1)
            ge = count_ids(mid, trips1, hi_c1, above1) >= _K
            return jnp.where(ge, mid, lo), jnp.where(ge, hi, mid - 1)

        lo, hi = lax.fori_loop(0, _TAIL1_ITERS, step_t1, (lo, hi))

        def rgrp(g, carry):
            cnt, abv = carry
            for j in range(_U):
                cid = ids[g * _U + j]
                v = bits_of(cid)
                pc = plsc.all_reduce_population_count(
                    jnp.logical_and(v >= lo, v <= hi))
                ids[cnt] = cid
                cnt = cnt + jnp.where(pc[0] > 0, 1, 0).astype(jnp.int32)
                abv = abv + jnp.logical_and(
                    v > hi, v <= hi_c1).astype(jnp.int32)
            return cnt, abv

        cnt2, above2_v = lax.fori_loop(0, trips1, rgrp,
                                       (jnp.int32(0), izeros))
        above2 = above1 + jnp.sum(above2_v)
        for j in range(_U):
            ids[cnt2 + j] = sent
        hi_c2 = hi
        trips2 = (cnt2 + _U - 1) // _U

        def step_t2(_, lohi):
            lo, hi = lohi
            mid = lo + ((hi - lo + 1) >> 1)
            ge = count_ids(mid, trips2, hi_c2, above2) >= _K
            return jnp.where(ge, mid, lo), jnp.where(ge, hi, mid - 1)

        lo, hi = lax.fori_loop(0, _TAIL2_ITERS, step_t2, (lo, hi))

        def egrp(g, dacc):
            for j in range(_U):
                c = g * _U + j
                v = sv[pl.ds(c * _L, _L)]
                bb = plsc.bitcast(v, jnp.int32)
                num = jnp.where(bb >= lo, jnp.exp(v), jnp.float32(0.0))
                nv[pl.ds(c * _L, _L)] = num
                dacc = dacc + num
            return dacc

        dacc = lax.fori_loop(0, nchunk // _U, egrp,
                             jnp.zeros((_L,), jnp.float32))
        den_v = jnp.broadcast_to(jnp.sum(dacc), (_L,))
        inv_v = jnp.ones((_L,), jnp.float32) / den_v

        def sgrp(g, carry):
            for j in range(_U):
                c = g * _U + j
                sv[pl.ds(c * _L, _L)] = nv[pl.ds(c * _L, _L)] * inv_v
            return carry

        lax.fori_loop(0, nchunk // _U, sgrp, 0)
        pltpu.sync_copy(sv.at[pl.ds(0, n)], o_hbm.at[b])

    nb = s_hbm.shape[0]
    for r in range(nb // 32):
        one_batch(wid + r * 32)


def _sc_stage2(scores):
    B, N = scores.shape
    mesh = plsc.VectorSubcoreMesh(
        core_axis_name="c", subcore_axis_name="s", num_cores=2,
        num_subcores=16)
    fn = pl.kernel(
        _sc_stage2_body,
        out_type=jax.ShapeDtypeStruct((B, N), jnp.float32),
        mesh=mesh,
        scratch_types=[
            pltpu.VMEM((N + _L,), jnp.float32),
            pltpu.VMEM((N,), jnp.float32),
            pltpu.SMEM((N // _L + _U,), jnp.int32),
        ],
        compiler_params=pltpu.CompilerParams(needs_layout_passes=False),
    )
    return fn(scores)


def kernel(x0, wa):
    B, N, D = x0.shape
    xt = jnp.transpose(x0, (0, 2, 1))
    wrow = wa.reshape(1, D)
    wbd = jnp.kron(jnp.eye(_BB, dtype=x0.dtype), wrow)

    half = B // 2
    outs = []
    for s in range(2):
        off = s * (half // _BB)
        scores = pl.pallas_call(
            _matvec_body,
            grid=(half // _BB,),
            in_specs=[
                pl.BlockSpec((_BB, D, N), lambda i, off=off: (i + off, 0, 0)),
                pl.BlockSpec((_BB, _BB * D), lambda i: (0, 0)),
            ],
            out_specs=pl.BlockSpec((_BB, N), lambda i: (i, 0)),
            out_shape=jax.ShapeDtypeStruct((half, N), jnp.float32),
        )(xt, wbd)
        outs.append(_sc_stage2(scores))

    out = jnp.concatenate(outs, axis=0)
    return out.reshape(B, N, 1)
